# Initial kernel scaffold; baseline (speedup 1.0000x reference)
import jax
import jax.numpy as jnp
from jax.experimental import pallas as pl


def kernel(nodes_feats, edge_index, W_self0, W_neigh0, b0, W_self1, W_neigh1, b1, W_self2, W_neigh2, b2, gamma0, beta0, gamma1, beta1):
    raise NotImplementedError("write your pallas kernel here")


# speedup vs baseline: 3.9386x; 3.9386x over previous
import functools

import jax
import jax.numpy as jnp
from jax import lax
from jax.experimental import pallas as pl
from jax.experimental.pallas import tpu as pltpu
from jax.experimental.pallas import tpu_sc as plsc

N_NODES = 10000
N_EDGES = 320000
D = 128

NC, NS = 2, 16
NW = NC * NS
CH = 128
N_PAD = 10112
E_PAD = ((N_EDGES + NW * CH - 1) // (NW * CH)) * NW * CH
EPW = E_PAD // NW
CHUNKS = EPW // CH
ROWS_PER_TILE = N_PAD // NS


_PIECES = []
_off = 0
while _off < ROWS_PER_TILE:
    _sz = min(CH, ROWS_PER_TILE - _off)
    _PIECES.append((_off, _sz))
    _off += _sz


def _sc_agg_body(z_hbm, src_hbm, dst_hbm, zeros_hbm, out_hbm,
                 sidx, didx, rows, shared, sem):
    cid = lax.axis_index("c")
    sid = lax.axis_index("s")
    wid = sid * NC + cid

    rb = ROWS_PER_TILE
    r0 = pl.multiple_of(sid * rb, 8)
    pltpu.sync_copy(zeros_hbm.at[pl.ds(0, CH)], rows)
    for off, sz in _PIECES:
        pltpu.sync_copy(rows.at[pl.ds(0, sz)], shared.at[pl.ds(r0 + off, sz)])
    plsc.subcore_barrier()

    def chunk(g, carry):
        eb = pl.multiple_of(wid * EPW + g * CH, 8)
        pltpu.sync_copy(src_hbm.at[pl.ds(eb, CH)], sidx)
        pltpu.sync_copy(dst_hbm.at[pl.ds(eb, CH)], didx)
        pltpu.async_copy(z_hbm.at[sidx], rows, sem).wait()
        pltpu.sync_copy(rows, shared.at[didx], add=True)
        return carry

    lax.fori_loop(0, CHUNKS, chunk, 0)
    plsc.subcore_barrier()

    o0 = pl.multiple_of(cid * N_PAD + sid * rb, 8)
    for off, sz in _PIECES:
        pltpu.sync_copy(shared.at[pl.ds(r0 + off, sz)], rows.at[pl.ds(0, sz)])
        pltpu.sync_copy(rows.at[pl.ds(0, sz)], out_hbm.at[pl.ds(o0 + off, sz)])


def _sc_deg_body(dst_hbm, zeros_hbm, ones_hbm, deg_hbm,
                 didx, dbuf, ones_v, shared_deg, sem):
    cid = lax.axis_index("c")
    sid = lax.axis_index("s")
    wid = sid * NC + cid

    rb = ROWS_PER_TILE
    r0 = pl.multiple_of(sid * rb, 8)
    pltpu.sync_copy(ones_hbm, ones_v)
    pltpu.sync_copy(zeros_hbm.at[pl.ds(0, CH)], dbuf)
    for off, sz in _PIECES:
        pltpu.sync_copy(dbuf.at[pl.ds(0, sz)],
                        shared_deg.at[pl.ds(r0 + off, sz)])
    plsc.subcore_barrier()

    def chunk(g, carry):
        eb = pl.multiple_of(wid * EPW + g * CH, 8)
        pltpu.sync_copy(dst_hbm.at[pl.ds(eb, CH)], didx)
        pltpu.sync_copy(ones_v, shared_deg.at[didx], add=True)
        return carry

    lax.fori_loop(0, CHUNKS, chunk, 0)
    plsc.subcore_barrier()

    o0 = pl.multiple_of(cid * N_PAD + sid * rb, 8)
    for off, sz in _PIECES:
        pltpu.sync_copy(shared_deg.at[pl.ds(r0 + off, sz)],
                        dbuf.at[pl.ds(0, sz)])
        pltpu.sync_copy(dbuf.at[pl.ds(0, sz)],
                        deg_hbm.at[pl.ds(o0 + off, sz)])


_MESH = plsc.VectorSubcoreMesh(core_axis_name="c", subcore_axis_name="s")

_sc_agg = pl.kernel(
    _sc_agg_body,
    out_type=[jax.ShapeDtypeStruct((NC * N_PAD, D), jnp.float32)],
    mesh=_MESH,
    scratch_types=[
        pltpu.VMEM((CH,), jnp.int32),
        pltpu.VMEM((CH,), jnp.int32),
        pltpu.VMEM((CH, D), jnp.float32),
        pltpu.VMEM_SHARED((N_PAD, D), jnp.float32),
        pltpu.SemaphoreType.DMA,
    ],
)

_sc_deg = pl.kernel(
    _sc_deg_body,
    out_type=[jax.ShapeDtypeStruct((NC * N_PAD, D), jnp.float32)],
    mesh=_MESH,
    scratch_types=[
        pltpu.VMEM((CH,), jnp.int32),
        pltpu.VMEM((CH, D), jnp.float32),
        pltpu.VMEM((CH, D), jnp.float32),
        pltpu.VMEM_SHARED((N_PAD, D), jnp.float32),
        pltpu.SemaphoreType.DMA,
    ],
)

BR = 1000


def _combine_body(last, x_ref, p0_ref, p1_ref, d0_ref, d1_ref, ws_ref, wn_ref,
                  b_ref, g_ref, be_ref, o_ref):
    deg = d0_ref[:, 0:1] + d1_ref[:, 0:1]
    agg = (p0_ref[...] + p1_ref[...]) / jnp.maximum(deg, 1.0)
    h = (jnp.dot(x_ref[...], ws_ref[...], preferred_element_type=jnp.float32)
         + jnp.dot(agg, wn_ref[...], preferred_element_type=jnp.float32)
         + b_ref[...])
    if last:
        m = jnp.max(h, axis=-1, keepdims=True)
        o_ref[...] = (h - m) - jnp.log(jnp.sum(jnp.exp(h - m), axis=-1,
                                               keepdims=True))
    else:
        mu = jnp.mean(h, axis=-1, keepdims=True)
        var = jnp.mean((h - mu) ** 2, axis=-1, keepdims=True)
        hn = (h - mu) * lax.rsqrt(var + 1e-5) * g_ref[...] + be_ref[...]
        o_ref[...] = jnp.maximum(hn, 0.0)


def _combine(x, parts, degparts, ws, wn, b, gamma, beta, last):
    p0 = parts[0:N_NODES]
    p1 = parts[N_PAD:N_PAD + N_NODES]
    d0 = degparts[0:N_NODES]
    d1 = degparts[N_PAD:N_PAD + N_NODES]
    b2 = b.reshape(1, D)
    g2 = gamma.reshape(1, D) if gamma is not None else b2
    be2 = beta.reshape(1, D) if beta is not None else b2
    grid = N_NODES // BR
    row_spec = pl.BlockSpec((BR, D), lambda i: (i, 0))
    deg_spec = pl.BlockSpec((BR, D), lambda i: (i, 0))
    w_spec = pl.BlockSpec((D, D), lambda i: (0, 0))
    v_spec = pl.BlockSpec((1, D), lambda i: (0, 0))
    return pl.pallas_call(
        functools.partial(_combine_body, last),
        grid=(grid,),
        in_specs=[row_spec, row_spec, row_spec, deg_spec, deg_spec,
                  w_spec, w_spec, v_spec, v_spec, v_spec],
        out_specs=row_spec,
        out_shape=jax.ShapeDtypeStruct((N_NODES, D), jnp.float32),
    )(x, p0, p1, d0, d1, ws, wn, b2, g2, be2)


def kernel(nodes_feats, edge_index, W_self0, W_neigh0, b0, W_self1, W_neigh1,
           b1, W_self2, W_neigh2, b2, gamma0, beta0, gamma1, beta1):
    src = edge_index[0]
    dst = edge_index[1]
    pad = E_PAD - N_EDGES
    srcp = jnp.concatenate([src, jnp.zeros((pad,), jnp.int32)])
    dstp = jnp.concatenate([dst, jnp.full((pad,), N_NODES, jnp.int32)])
    zeros = jnp.zeros((N_PAD, D), jnp.float32)
    ones = jnp.ones((CH, D), jnp.float32)

    degparts, = _sc_deg(dstp, zeros, ones)
    parts0, = _sc_agg(nodes_feats, srcp, dstp, zeros)
    h = _combine(nodes_feats, parts0, degparts, W_self0, W_neigh0, b0,
                 gamma0, beta0, last=False)
    parts1, = _sc_agg(h, srcp, dstp, zeros)
    h = _combine(h, parts1, degparts, W_self1, W_neigh1, b1,
                 gamma1, beta1, last=False)
    parts2, = _sc_agg(h, srcp, dstp, zeros)
    return _combine(h, parts2, degparts, W_self2, W_neigh2, b2,
                    None, None, last=True)
